# Initial kernel scaffold; baseline (speedup 1.0000x reference)
import jax
import jax.numpy as jnp
from jax.experimental import pallas as pl


def kernel(user_x, movie_x, edge_index, edge_attr, W_u, b_u, W_m, b_m, W1, b1, W2, b2):
    raise NotImplementedError("write your pallas kernel here")


# speedup vs baseline: 15.0177x; 15.0177x over previous
import functools

import jax
import jax.numpy as jnp
from jax import lax
from jax.experimental import pallas as pl
from jax.experimental.pallas import tpu as pltpu
from jax.experimental.pallas import tpu_sc as plsc

N_NODES = 100000
HIDDEN = 64

NS = 16
STRIPE = 6272
NPAD = NS * STRIPE
KB = 128


def _pad_up(x, m):
    return ((x + m - 1) // m) * m


def _deg_body(col_h, ew_h, out_h, colv, ewv, zrow, acc):
    cid = lax.axis_index("c")
    sid = lax.axis_index("s")
    epad = col_h.shape[0]
    per_tile = epad // 32
    nb = per_tile // KB

    def zero_z(i, c):
        zrow[pl.ds(i * 16, 16)] = jnp.zeros((16,), jnp.float32)
        return c

    lax.fori_loop(0, STRIPE // 16, zero_z, 0)
    pltpu.sync_copy(zrow, acc.at[pl.ds(sid * STRIPE, STRIPE)])
    plsc.subcore_barrier()

    ebase = cid * (epad // 2) + sid * per_tile

    def body(b, c):
        off = ebase + b * KB
        pltpu.sync_copy(col_h.at[pl.ds(off, KB)], colv)
        pltpu.sync_copy(ew_h.at[pl.ds(off, KB)], ewv)
        pltpu.sync_copy(ewv, acc.at[colv], add=True)
        return c

    lax.fori_loop(0, nb, body, 0)
    plsc.subcore_barrier()
    pltpu.sync_copy(acc.at[pl.ds(sid * STRIPE, STRIPE)],
                    out_h.at[cid, pl.ds(sid * STRIPE, STRIPE)])


def _deg_call(colp, ewp):
    mesh = plsc.VectorSubcoreMesh(core_axis_name="c", subcore_axis_name="s")
    return pl.kernel(
        _deg_body,
        out_type=jax.ShapeDtypeStruct((2, NPAD), jnp.float32),
        mesh=mesh,
        scratch_types=[
            pltpu.VMEM((KB,), jnp.int32),
            pltpu.VMEM((KB,), jnp.float32),
            pltpu.VMEM((STRIPE,), jnp.float32),
            pltpu.VMEM_SHARED((NPAD,), jnp.float32),
        ],
    )(colp, ewp)


def _agg_body(qa_h, qb_h, row_h, col_h, ew_h, outa_h, outb_h,
              rowv, colv, ewv, rows, sem, acc):
    cid = lax.axis_index("c")
    sid = lax.axis_index("s")
    epad = row_h.shape[0]
    per_tile = epad // 16
    nb = per_tile // KB

    def zero_r(i, c):
        rows[i, :] = jnp.zeros((16,), jnp.float32)
        return c

    lax.fori_loop(0, KB, zero_r, 0)

    def zero_acc(i, c):
        pltpu.sync_copy(rows, acc.at[pl.ds(sid * STRIPE + i * KB, KB)])
        return c

    lax.fori_loop(0, STRIPE // KB, zero_acc, 0)
    plsc.subcore_barrier()

    ebase = sid * per_tile

    def body(b, c):
        off = ebase + b * KB
        pltpu.sync_copy(row_h.at[pl.ds(off, KB)], rowv)
        pltpu.sync_copy(col_h.at[pl.ds(off, KB)], colv)
        pltpu.sync_copy(ew_h.at[pl.ds(off, KB)], ewv)

        @pl.when(cid == 0)
        def _():
            pltpu.async_copy(qa_h.at[rowv], rows, sem).wait()

        @pl.when(cid == 1)
        def _():
            pltpu.async_copy(qb_h.at[rowv], rows, sem).wait()

        def scale16(j, c2):
            wv = ewv[pl.ds(j * 16, 16)]
            for l in range(16):
                w = jnp.broadcast_to(wv[l], (16,))
                rows[j * 16 + l, :] = rows[j * 16 + l, :] * w
            return c2

        lax.fori_loop(0, KB // 16, scale16, 0)
        pltpu.sync_copy(rows, acc.at[colv], add=True)
        return c

    lax.fori_loop(0, nb, body, 0)
    plsc.subcore_barrier()

    @pl.when(cid == 0)
    def _():
        pltpu.sync_copy(acc.at[pl.ds(sid * STRIPE, STRIPE)],
                        outa_h.at[pl.ds(sid * STRIPE, STRIPE)])

    @pl.when(cid == 1)
    def _():
        pltpu.sync_copy(acc.at[pl.ds(sid * STRIPE, STRIPE)],
                        outb_h.at[pl.ds(sid * STRIPE, STRIPE)])


def _agg_call(qa, qb, rowp, colp, ewp):
    mesh = plsc.VectorSubcoreMesh(core_axis_name="c", subcore_axis_name="s")
    return pl.kernel(
        _agg_body,
        out_type=[jax.ShapeDtypeStruct((NPAD, 16), jnp.float32),
                  jax.ShapeDtypeStruct((NPAD, 16), jnp.float32)],
        mesh=mesh,
        scratch_types=[
            pltpu.VMEM((KB,), jnp.int32),
            pltpu.VMEM((KB,), jnp.int32),
            pltpu.VMEM((KB,), jnp.float32),
            pltpu.VMEM((KB, 16), jnp.float32),
            pltpu.SemaphoreType.DMA,
            pltpu.VMEM_SHARED((NPAD, 16), jnp.float32),
        ],
        compiler_params=pltpu.CompilerParams(use_tc_tiling_on_sc=False),
    )(qa, qb, rowp, colp, ewp)


_BN = 2000


def _prep_body(u_ref, m_ref, d0_ref, d1_ref, qa_ref, qb_ref, dinv_ref):
    deg = d0_ref[...] + d1_ref[...] + 1.0
    dinv = lax.rsqrt(deg)
    u = u_ref[...]
    m = m_ref[...]
    qa_ref[...] = jnp.concatenate([u, m[:, :13]], axis=1) * dinv
    qb_ref[...] = jnp.concatenate(
        [m[:, 13:], jnp.zeros((_BN, 11), jnp.float32)], axis=1) * dinv
    dinv_ref[...] = dinv


def _prep_call(user_x, movie_x, d0, d1):
    grid = (N_NODES // _BN,)
    row_spec = lambda w: pl.BlockSpec((_BN, w), lambda i: (i, 0))
    return pl.pallas_call(
        _prep_body,
        grid=grid,
        in_specs=[row_spec(3), row_spec(18), row_spec(1), row_spec(1)],
        out_specs=[row_spec(16), row_spec(16), row_spec(1)],
        out_shape=[jax.ShapeDtypeStruct((N_NODES, 16), jnp.float32),
                   jax.ShapeDtypeStruct((N_NODES, 16), jnp.float32),
                   jax.ShapeDtypeStruct((N_NODES, 1), jnp.float32)],
    )(user_x, movie_x, d0, d1)


def _final_body(aa_ref, ab_ref, u_ref, m_ref, dinv_ref,
                wz_ref, bz_ref, w2_ref, b2_ref, o_ref):
    dinv = dinv_ref[...]
    agg = jnp.concatenate([aa_ref[...], ab_ref[...]], axis=1)
    x32 = jnp.concatenate(
        [u_ref[...], m_ref[...], jnp.zeros((_BN, 11), jnp.float32)], axis=1)
    z = dinv * agg + (dinv * dinv) * x32
    hh = jnp.maximum(
        jnp.dot(z, wz_ref[...], preferred_element_type=jnp.float32)
        + bz_ref[...], 0.0)
    o_ref[...] = (jnp.dot(hh, w2_ref[...], preferred_element_type=jnp.float32)
                  + b2_ref[...])


def _final_call(aa, ab, user_x, movie_x, dinv, wz, bz, w2, b2):
    grid = (N_NODES // _BN,)
    row_spec = lambda w: pl.BlockSpec((_BN, w), lambda i: (i, 0))
    full = lambda a, b: pl.BlockSpec((a, b), lambda i: (0, 0))
    return pl.pallas_call(
        _final_body,
        grid=grid,
        in_specs=[row_spec(16), row_spec(16), row_spec(3), row_spec(18),
                  row_spec(1), full(32, 128), full(1, 128), full(128, 1),
                  full(1, 1)],
        out_specs=row_spec(1),
        out_shape=jax.ShapeDtypeStruct((N_NODES, 1), jnp.float32),
    )(aa, ab, user_x, movie_x, dinv, wz, bz, w2, b2)


def kernel(user_x, movie_x, edge_index, edge_attr,
           W_u, b_u, W_m, b_m, W1, b1, W2, b2):
    E = edge_attr.shape[0]
    epad = _pad_up(E, 4096)
    pad = epad - E

    row = edge_index[0].astype(jnp.int32)
    col = edge_index[1].astype(jnp.int32)
    zi = jnp.zeros((pad,), jnp.int32)
    rowp = jnp.concatenate([row, zi])
    colp = jnp.concatenate([col, zi])
    ewp = jnp.concatenate([edge_attr, jnp.zeros((pad,), jnp.float32)])

    Wcat = jnp.zeros((32, 2 * HIDDEN), jnp.float32)
    Wcat = Wcat.at[0:3, 0:HIDDEN].set(W_u)
    Wcat = Wcat.at[3:21, HIDDEN:2 * HIDDEN].set(W_m)
    bcat = jnp.concatenate([b_u, b_m])
    Wz = Wcat @ W1
    bz = (bcat @ W1 + b1).reshape(1, -1)

    degp = _deg_call(colp, ewp)
    d0 = degp[0, :N_NODES].reshape(N_NODES, 1)
    d1 = degp[1, :N_NODES].reshape(N_NODES, 1)

    qa, qb, dinv = _prep_call(user_x, movie_x, d0, d1)
    acca, accb = _agg_call(qa, qb, rowp, colp, ewp)

    return _final_call(acca[:N_NODES], accb[:N_NODES], user_x, movie_x,
                       dinv, Wz, bz, W2, b2.reshape(1, 1))


# speedup vs baseline: 45.1568x; 3.0069x over previous
import jax
import jax.numpy as jnp
from jax import lax
from jax.experimental import pallas as pl
from jax.experimental.pallas import tpu as pltpu
from jax.experimental.pallas import tpu_sc as plsc

N_NODES = 100000
HIDDEN = 64

NS = 16
STRIPE = 6272
NPAD = NS * STRIPE
KB = 128
ECHUNK = 512
DCHUNK = 1024


def _pad_up(x, m):
    return ((x + m - 1) // m) * m


def _mesh():
    return plsc.VectorSubcoreMesh(core_axis_name="c", subcore_axis_name="s")


def _deg_body(colb_h, ew_h, out_h, colc0, colc1, ewc0, ewc1, zrow, acc,
              sl0, sl1, ss0, ss1):
    cid = lax.axis_index("c")
    sid = lax.axis_index("s")
    epad = ew_h.shape[0]
    per_tile = epad // 32
    nch = per_tile // DCHUNK
    ebase = cid * (epad // 2) + sid * per_tile
    rbase = ebase // KB
    colc = (colc0, colc1)
    ewc = (ewc0, ewc1)
    sl = (sl0, sl1)
    ss = (ss0, ss1)
    nbat = DCHUNK // KB

    def lin_issue(p, c):
        off = ebase + c * DCHUNK
        pltpu.async_copy(colb_h.at[pl.ds(rbase + c * nbat, nbat)],
                         colc[p], sl[p])
        pltpu.async_copy(ew_h.at[pl.ds(off, DCHUNK)], ewc[p], sl[p])

    def lin_drain(p):
        pltpu.make_async_copy(colb_h.at[pl.ds(0, nbat)], colc[p], sl[p]).wait()
        pltpu.make_async_copy(ew_h.at[pl.ds(0, DCHUNK)], ewc[p], sl[p]).wait()

    def s_fire(p):
        for s in range(nbat):
            pltpu.async_copy(ewc[p].at[pl.ds(s * KB, KB)],
                             acc.at[colc[p].at[s]], ss[p], add=True)

    def s_drain(p):
        for s in range(nbat):
            pltpu.make_async_copy(ewc[p].at[pl.ds(s * KB, KB)],
                                  acc.at[pl.ds(0, KB)], ss[p]).wait()

    lin_issue(0, 0)
    lin_issue(1, 1)

    def zz(i, c):
        zrow[pl.ds(i * 16, 16)] = jnp.zeros((16,), jnp.float32)
        return c

    lax.fori_loop(0, STRIPE // 16, zz, 0)
    pltpu.sync_copy(zrow, acc.at[pl.ds(sid * STRIPE, STRIPE)])
    plsc.subcore_barrier()

    npair = (nch - 1) // 2

    def pair(i, c):
        a = 2 * i
        lin_drain(0)
        s_fire(0)
        lin_drain(1)
        s_fire(1)
        s_drain(0)
        lin_issue(0, a + 2)
        s_drain(1)

        @pl.when(i < npair - 1)
        def _():
            lin_issue(1, a + 3)

        return c

    lax.fori_loop(0, npair, pair, 0)
    lin_drain(0)
    s_fire(0)
    s_drain(0)
    plsc.subcore_barrier()
    pltpu.sync_copy(acc.at[pl.ds(sid * STRIPE, STRIPE)],
                    out_h.at[cid, pl.ds(sid * STRIPE, STRIPE)])


def _deg_call(colb, ewp):
    return pl.kernel(
        _deg_body,
        out_type=jax.ShapeDtypeStruct((2, NPAD), jnp.float32),
        mesh=_mesh(),
        scratch_types=[
            pltpu.VMEM((DCHUNK // KB, KB), jnp.int32),
            pltpu.VMEM((DCHUNK // KB, KB), jnp.int32),
            pltpu.VMEM((DCHUNK,), jnp.float32),
            pltpu.VMEM((DCHUNK,), jnp.float32),
            pltpu.VMEM((STRIPE,), jnp.float32),
            pltpu.VMEM_SHARED((NPAD,), jnp.float32),
            pltpu.SemaphoreType.DMA,
            pltpu.SemaphoreType.DMA,
            pltpu.SemaphoreType.DMA,
            pltpu.SemaphoreType.DMA,
        ],
        compiler_params=pltpu.CompilerParams(use_tc_tiling_on_sc=False),
    )(colb, ewp)


def _agg_body(qa_h, qb_h, row_h, colb_h, ew_h, outa_h, outb_h,
              rowc0, rowc1, colc0, colc1, ewc0, ewc1, rows0, rows1, acc,
              sl0, sl1, sg0, sg1, ss0, ss1):
    cid = lax.axis_index("c")
    sid = lax.axis_index("s")
    epad = ew_h.shape[0]
    per_tile = epad // 16
    nch = per_tile // ECHUNK
    odd = (nch % 2 == 1)
    nch_main = nch if odd else nch - 1
    ebase = sid * per_tile
    rbase = ebase // KB
    rowc = (rowc0, rowc1)
    colc = (colc0, colc1)
    ewc = (ewc0, ewc1)
    rows = (rows0, rows1)
    sl = (sl0, sl1)
    sg = (sg0, sg1)
    ss = (ss0, ss1)
    nbat = ECHUNK // KB

    def lin_issue(p, c):
        off = ebase + c * ECHUNK
        pltpu.async_copy(row_h.at[pl.ds(off, ECHUNK)], rowc[p], sl[p])
        pltpu.async_copy(colb_h.at[pl.ds(rbase + c * nbat, nbat)],
                         colc[p], sl[p])
        pltpu.async_copy(ew_h.at[pl.ds(off, ECHUNK)], ewc[p], sl[p])

    def lin_drain(p):
        pltpu.make_async_copy(row_h.at[pl.ds(0, ECHUNK)], rowc[p], sl[p]).wait()
        pltpu.make_async_copy(colb_h.at[pl.ds(0, nbat)], colc[p], sl[p]).wait()
        pltpu.make_async_copy(ew_h.at[pl.ds(0, ECHUNK)], ewc[p], sl[p]).wait()

    def g_fire(p):
        @pl.when(cid == 0)
        def _():
            for s in range(nbat):
                pltpu.async_copy(qa_h.at[rowc[p].at[pl.ds(s * KB, KB)]],
                                 rows[p].at[pl.ds(s * KB, KB)], sg[p])

        @pl.when(cid == 1)
        def _():
            for s in range(nbat):
                pltpu.async_copy(qb_h.at[rowc[p].at[pl.ds(s * KB, KB)]],
                                 rows[p].at[pl.ds(s * KB, KB)], sg[p])

    def g_drain(p):
        for s in range(nbat):
            pltpu.make_async_copy(qa_h.at[pl.ds(0, KB)],
                                  rows[p].at[pl.ds(s * KB, KB)], sg[p]).wait()

    def scale(p):
        rr = rows[p]
        ee = ewc[p]

        def grp(j, c):
            wv = ee[pl.ds(j * 16, 16)]
            for l in range(16):
                w = jnp.broadcast_to(wv[l], (16,))
                rr[j * 16 + l, :] = rr[j * 16 + l, :] * w
            return c

        lax.fori_loop(0, ECHUNK // 16, grp, 0)

    def s_fire(p):
        for s in range(nbat):
            pltpu.async_copy(rows[p].at[pl.ds(s * KB, KB)],
                             acc.at[colc[p].at[s]], ss[p], add=True)

    def s_drain(p):
        for s in range(nbat):
            pltpu.make_async_copy(rows[p].at[pl.ds(s * KB, KB)],
                                  acc.at[pl.ds(0, KB)], ss[p]).wait()

    lin_issue(0, 0)
    lin_issue(1, 1)

    def zz(i, c):
        rows0[i, :] = jnp.zeros((16,), jnp.float32)
        return c

    lax.fori_loop(0, ECHUNK, zz, 0)
    for k in range(STRIPE // ECHUNK):
        pltpu.sync_copy(rows0, acc.at[pl.ds(sid * STRIPE + k * ECHUNK, ECHUNK)])
    rem = STRIPE % ECHUNK
    if rem:
        pltpu.sync_copy(rows0.at[pl.ds(0, rem)],
                        acc.at[pl.ds(sid * STRIPE + (STRIPE - rem), rem)])
    plsc.subcore_barrier()

    lin_drain(0)
    g_fire(0)
    lin_drain(1)
    g_fire(1)

    npair = (nch_main - 1) // 2

    def pair(i, c):
        a = 2 * i
        g_drain(0)
        scale(0)
        s_fire(0)
        g_drain(1)
        scale(1)
        s_fire(1)
        s_drain(0)
        lin_issue(0, a + 2)
        s_drain(1)

        @pl.when(i < npair - 1)
        def _():
            lin_issue(1, a + 3)

        lin_drain(0)
        g_fire(0)

        @pl.when(i < npair - 1)
        def _():
            lin_drain(1)
            g_fire(1)

        return c

    lax.fori_loop(0, npair, pair, 0)
    g_drain(0)
    scale(0)
    s_fire(0)
    s_drain(0)
    if not odd:
        lin_issue(1, nch - 1)
        lin_drain(1)
        g_fire(1)
        g_drain(1)
        scale(1)
        s_fire(1)
        s_drain(1)
    plsc.subcore_barrier()

    @pl.when(cid == 0)
    def _():
        pltpu.sync_copy(acc.at[pl.ds(sid * STRIPE, STRIPE)],
                        outa_h.at[pl.ds(sid * STRIPE, STRIPE)])

    @pl.when(cid == 1)
    def _():
        pltpu.sync_copy(acc.at[pl.ds(sid * STRIPE, STRIPE)],
                        outb_h.at[pl.ds(sid * STRIPE, STRIPE)])


def _agg_call(qa, qb, rowp, colb, ewp):
    return pl.kernel(
        _agg_body,
        out_type=[jax.ShapeDtypeStruct((NPAD, 16), jnp.float32),
                  jax.ShapeDtypeStruct((NPAD, 16), jnp.float32)],
        mesh=_mesh(),
        scratch_types=[
            pltpu.VMEM((ECHUNK,), jnp.int32),
            pltpu.VMEM((ECHUNK,), jnp.int32),
            pltpu.VMEM((ECHUNK // KB, KB), jnp.int32),
            pltpu.VMEM((ECHUNK // KB, KB), jnp.int32),
            pltpu.VMEM((ECHUNK,), jnp.float32),
            pltpu.VMEM((ECHUNK,), jnp.float32),
            pltpu.VMEM((ECHUNK, 16), jnp.float32),
            pltpu.VMEM((ECHUNK, 16), jnp.float32),
            pltpu.VMEM_SHARED((NPAD, 16), jnp.float32),
            pltpu.SemaphoreType.DMA,
            pltpu.SemaphoreType.DMA,
            pltpu.SemaphoreType.DMA,
            pltpu.SemaphoreType.DMA,
            pltpu.SemaphoreType.DMA,
            pltpu.SemaphoreType.DMA,
        ],
        compiler_params=pltpu.CompilerParams(use_tc_tiling_on_sc=False),
    )(qa, qb, rowp, colb, ewp)


_BN = 2000


def _prep_body(u_ref, m_ref, d0_ref, d1_ref, qa_ref, qb_ref, dinv_ref):
    deg = d0_ref[...] + d1_ref[...] + 1.0
    dinv = lax.rsqrt(deg)
    u = u_ref[...]
    m = m_ref[...]
    qa_ref[...] = jnp.concatenate([u, m[:, :13]], axis=1) * dinv
    qb_ref[...] = jnp.concatenate(
        [m[:, 13:], jnp.zeros((_BN, 11), jnp.float32)], axis=1) * dinv
    dinv_ref[...] = dinv


def _prep_call(user_x, movie_x, d0, d1):
    grid = (N_NODES // _BN,)
    row_spec = lambda w: pl.BlockSpec((_BN, w), lambda i: (i, 0))
    return pl.pallas_call(
        _prep_body,
        grid=grid,
        in_specs=[row_spec(3), row_spec(18), row_spec(1), row_spec(1)],
        out_specs=[row_spec(16), row_spec(16), row_spec(1)],
        out_shape=[jax.ShapeDtypeStruct((N_NODES, 16), jnp.float32),
                   jax.ShapeDtypeStruct((N_NODES, 16), jnp.float32),
                   jax.ShapeDtypeStruct((N_NODES, 1), jnp.float32)],
    )(user_x, movie_x, d0, d1)


def _final_body(aa_ref, ab_ref, u_ref, m_ref, dinv_ref,
                wz_ref, bz_ref, w2_ref, b2_ref, o_ref):
    dinv = dinv_ref[...]
    agg = jnp.concatenate([aa_ref[...], ab_ref[...]], axis=1)
    x32 = jnp.concatenate(
        [u_ref[...], m_ref[...], jnp.zeros((_BN, 11), jnp.float32)], axis=1)
    z = dinv * agg + (dinv * dinv) * x32
    hh = jnp.maximum(
        jnp.dot(z, wz_ref[...], preferred_element_type=jnp.float32)
        + bz_ref[...], 0.0)
    o_ref[...] = (jnp.dot(hh, w2_ref[...], preferred_element_type=jnp.float32)
                  + b2_ref[...])


def _final_call(aa, ab, user_x, movie_x, dinv, wz, bz, w2, b2):
    grid = (N_NODES // _BN,)
    row_spec = lambda w: pl.BlockSpec((_BN, w), lambda i: (i, 0))
    full = lambda a, b: pl.BlockSpec((a, b), lambda i: (0, 0))
    return pl.pallas_call(
        _final_body,
        grid=grid,
        in_specs=[row_spec(16), row_spec(16), row_spec(3), row_spec(18),
                  row_spec(1), full(32, 128), full(1, 128), full(128, 1),
                  full(1, 1)],
        out_specs=row_spec(1),
        out_shape=jax.ShapeDtypeStruct((N_NODES, 1), jnp.float32),
    )(aa, ab, user_x, movie_x, dinv, wz, bz, w2, b2)


def kernel(user_x, movie_x, edge_index, edge_attr,
           W_u, b_u, W_m, b_m, W1, b1, W2, b2):
    E = edge_attr.shape[0]
    epad = _pad_up(E, 32 * DCHUNK)
    pad = epad - E

    row = edge_index[0].astype(jnp.int32)
    col = edge_index[1].astype(jnp.int32)
    zi = jnp.zeros((pad,), jnp.int32)
    rowp = jnp.concatenate([row, zi])
    colp = jnp.concatenate([col, zi])
    colb = colp.reshape(epad // KB, KB)
    ewp = jnp.concatenate([edge_attr, jnp.zeros((pad,), jnp.float32)])

    Wcat = jnp.zeros((32, 2 * HIDDEN), jnp.float32)
    Wcat = Wcat.at[0:3, 0:HIDDEN].set(W_u)
    Wcat = Wcat.at[3:21, HIDDEN:2 * HIDDEN].set(W_m)
    bcat = jnp.concatenate([b_u, b_m])
    Wz = Wcat @ W1
    bz = (bcat @ W1 + b1).reshape(1, -1)

    degp = _deg_call(colb, ewp)
    d0 = degp[0, :N_NODES].reshape(N_NODES, 1)
    d1 = degp[1, :N_NODES].reshape(N_NODES, 1)

    qa, qb, dinv = _prep_call(user_x, movie_x, d0, d1)
    acca, accb = _agg_call(qa, qb, rowp, colb, ewp)

    return _final_call(acca[:N_NODES], accb[:N_NODES], user_x, movie_x,
                       dinv, Wz, bz, W2, b2.reshape(1, 1))


# speedup vs baseline: 49.4690x; 1.0955x over previous
import jax
import jax.numpy as jnp
from jax import lax
from jax.experimental import pallas as pl
from jax.experimental.pallas import tpu as pltpu
from jax.experimental.pallas import tpu_sc as plsc

N_NODES = 100000
HIDDEN = 64

NS = 16
STRIPE = 6272
NPAD = NS * STRIPE
KB = 128
ECHUNK = 512
DCHUNK = 1024


def _pad_up(x, m):
    return ((x + m - 1) // m) * m


def _mesh():
    return plsc.VectorSubcoreMesh(core_axis_name="c", subcore_axis_name="s")


def _deg_body(colb_h, ew_h, out_h, colc0, colc1, ewc0, ewc1, zrow, acc,
              sl0, sl1, ss0, ss1):
    cid = lax.axis_index("c")
    sid = lax.axis_index("s")
    epad = ew_h.shape[0]
    per_tile = epad // 32
    nch = per_tile // DCHUNK
    ebase = cid * (epad // 2) + sid * per_tile
    rbase = ebase // KB
    colc = (colc0, colc1)
    ewc = (ewc0, ewc1)
    sl = (sl0, sl1)
    ss = (ss0, ss1)
    nbat = DCHUNK // KB

    def lin_issue(p, c):
        off = ebase + c * DCHUNK
        pltpu.async_copy(colb_h.at[pl.ds(rbase + c * nbat, nbat)],
                         colc[p], sl[p])
        pltpu.async_copy(ew_h.at[pl.ds(off, DCHUNK)], ewc[p], sl[p])

    def lin_drain(p):
        pltpu.make_async_copy(colb_h.at[pl.ds(0, nbat)], colc[p], sl[p]).wait()
        pltpu.make_async_copy(ew_h.at[pl.ds(0, DCHUNK)], ewc[p], sl[p]).wait()

    def s_fire(p):
        for s in range(nbat):
            pltpu.async_copy(ewc[p].at[pl.ds(s * KB, KB)],
                             acc.at[colc[p].at[s]], ss[p], add=True)

    def s_drain(p):
        for s in range(nbat):
            pltpu.make_async_copy(ewc[p].at[pl.ds(s * KB, KB)],
                                  acc.at[pl.ds(0, KB)], ss[p]).wait()

    lin_issue(0, 0)
    lin_issue(1, 1)

    def zz(i, c):
        zrow[pl.ds(i * 16, 16)] = jnp.zeros((16,), jnp.float32)
        return c

    lax.fori_loop(0, STRIPE // 16, zz, 0)
    pltpu.sync_copy(zrow, acc.at[pl.ds(sid * STRIPE, STRIPE)])
    plsc.subcore_barrier()

    npair = (nch - 1) // 2

    def pair(i, c):
        a = 2 * i
        lin_drain(0)
        s_fire(0)
        lin_drain(1)
        s_fire(1)
        s_drain(0)
        lin_issue(0, a + 2)
        s_drain(1)

        @pl.when(i < npair - 1)
        def _():
            lin_issue(1, a + 3)

        return c

    lax.fori_loop(0, npair, pair, 0)
    lin_drain(0)
    s_fire(0)
    s_drain(0)
    plsc.subcore_barrier()
    pltpu.sync_copy(acc.at[pl.ds(sid * STRIPE, STRIPE)],
                    out_h.at[cid, pl.ds(sid * STRIPE, STRIPE)])


def _deg_call(colb, ewp):
    return pl.kernel(
        _deg_body,
        out_type=jax.ShapeDtypeStruct((2, NPAD), jnp.float32),
        mesh=_mesh(),
        scratch_types=[
            pltpu.VMEM((DCHUNK // KB, KB), jnp.int32),
            pltpu.VMEM((DCHUNK // KB, KB), jnp.int32),
            pltpu.VMEM((DCHUNK,), jnp.float32),
            pltpu.VMEM((DCHUNK,), jnp.float32),
            pltpu.VMEM((STRIPE,), jnp.float32),
            pltpu.VMEM_SHARED((NPAD,), jnp.float32),
            pltpu.SemaphoreType.DMA,
            pltpu.SemaphoreType.DMA,
            pltpu.SemaphoreType.DMA,
            pltpu.SemaphoreType.DMA,
        ],
        compiler_params=pltpu.CompilerParams(use_tc_tiling_on_sc=False),
    )(colb, ewp)


def _agg_body(qa_h, qb_h, row_h, colb_h, ew_h, outa_h, outb_h,
              rowc0, rowc1, colc0, colc1, ewc0, ewc1, rows0, rows1, acc,
              sl0, sl1, sg0, sg1, ss0, ss1):
    cid = lax.axis_index("c")
    sid = lax.axis_index("s")
    epad = ew_h.shape[0]
    per_tile = epad // 16
    nch = per_tile // ECHUNK
    odd = (nch % 2 == 1)
    nch_main = nch if odd else nch - 1
    ebase = sid * per_tile
    rbase = ebase // KB
    rowc = (rowc0, rowc1)
    colc = (colc0, colc1)
    ewc = (ewc0, ewc1)
    rows = (rows0, rows1)
    sl = (sl0, sl1)
    sg = (sg0, sg1)
    ss = (ss0, ss1)
    nbat = ECHUNK // KB

    def lin_issue(p, c):
        off = ebase + c * ECHUNK
        pltpu.async_copy(row_h.at[pl.ds(off, ECHUNK)], rowc[p], sl[p])
        pltpu.async_copy(colb_h.at[pl.ds(rbase + c * nbat, nbat)],
                         colc[p], sl[p])
        pltpu.async_copy(ew_h.at[pl.ds(off, ECHUNK)], ewc[p], sl[p])

    def lin_drain(p):
        pltpu.make_async_copy(row_h.at[pl.ds(0, ECHUNK)], rowc[p], sl[p]).wait()
        pltpu.make_async_copy(colb_h.at[pl.ds(0, nbat)], colc[p], sl[p]).wait()
        pltpu.make_async_copy(ew_h.at[pl.ds(0, ECHUNK)], ewc[p], sl[p]).wait()

    def g_fire(p):
        @pl.when(cid == 0)
        def _():
            for s in range(nbat):
                pltpu.async_copy(qa_h.at[rowc[p].at[pl.ds(s * KB, KB)]],
                                 rows[p].at[pl.ds(s * KB, KB)], sg[p])

        @pl.when(cid == 1)
        def _():
            for s in range(nbat):
                pltpu.async_copy(qb_h.at[rowc[p].at[pl.ds(s * KB, KB)]],
                                 rows[p].at[pl.ds(s * KB, KB)], sg[p])

    def g_drain(p):
        for s in range(nbat):
            pltpu.make_async_copy(qa_h.at[pl.ds(0, KB)],
                                  rows[p].at[pl.ds(s * KB, KB)], sg[p]).wait()

    def scale(p):
        rr = rows[p]
        ee = ewc[p]

        def grp(j, c):
            wv = ee[pl.ds(j * 16, 16)]
            for l in range(16):
                w = jnp.broadcast_to(wv[l], (16,))
                rr[j * 16 + l, :] = rr[j * 16 + l, :] * w
            return c

        lax.fori_loop(0, ECHUNK // 16, grp, 0)

    def s_fire(p):
        for s in range(nbat):
            pltpu.async_copy(rows[p].at[pl.ds(s * KB, KB)],
                             acc.at[colc[p].at[s]], ss[p], add=True)

    def s_drain(p):
        for s in range(nbat):
            pltpu.make_async_copy(rows[p].at[pl.ds(s * KB, KB)],
                                  acc.at[pl.ds(0, KB)], ss[p]).wait()

    lin_issue(0, 0)
    lin_issue(1, 1)

    def zz(i, c):
        rows0[i, :] = jnp.zeros((16,), jnp.float32)
        return c

    lax.fori_loop(0, ECHUNK, zz, 0)
    for k in range(STRIPE // ECHUNK):
        pltpu.sync_copy(rows0, acc.at[pl.ds(sid * STRIPE + k * ECHUNK, ECHUNK)])
    rem = STRIPE % ECHUNK
    if rem:
        pltpu.sync_copy(rows0.at[pl.ds(0, rem)],
                        acc.at[pl.ds(sid * STRIPE + (STRIPE - rem), rem)])
    plsc.subcore_barrier()

    lin_drain(0)
    g_fire(0)
    lin_drain(1)
    g_fire(1)

    npair = (nch_main - 1) // 2

    def pair(i, c):
        a = 2 * i
        g_drain(0)
        scale(0)
        s_fire(0)
        g_drain(1)
        scale(1)
        s_fire(1)
        s_drain(0)
        lin_issue(0, a + 2)
        s_drain(1)

        @pl.when(i < npair - 1)
        def _():
            lin_issue(1, a + 3)

        lin_drain(0)
        g_fire(0)

        @pl.when(i < npair - 1)
        def _():
            lin_drain(1)
            g_fire(1)

        return c

    lax.fori_loop(0, npair, pair, 0)
    g_drain(0)
    scale(0)
    s_fire(0)
    s_drain(0)
    if not odd:
        lin_issue(1, nch - 1)
        lin_drain(1)
        g_fire(1)
        g_drain(1)
        scale(1)
        s_fire(1)
        s_drain(1)
    plsc.subcore_barrier()

    @pl.when(cid == 0)
    def _():
        pltpu.sync_copy(acc.at[pl.ds(sid * STRIPE, STRIPE)],
                        outa_h.at[pl.ds(sid * STRIPE, STRIPE)])

    @pl.when(cid == 1)
    def _():
        pltpu.sync_copy(acc.at[pl.ds(sid * STRIPE, STRIPE)],
                        outb_h.at[pl.ds(sid * STRIPE, STRIPE)])


def _agg_call(qa, qb, rowp, colb, ewp):
    return pl.kernel(
        _agg_body,
        out_type=[jax.ShapeDtypeStruct((NPAD, 16), jnp.float32),
                  jax.ShapeDtypeStruct((NPAD, 16), jnp.float32)],
        mesh=_mesh(),
        scratch_types=[
            pltpu.VMEM((ECHUNK,), jnp.int32),
            pltpu.VMEM((ECHUNK,), jnp.int32),
            pltpu.VMEM((ECHUNK // KB, KB), jnp.int32),
            pltpu.VMEM((ECHUNK // KB, KB), jnp.int32),
            pltpu.VMEM((ECHUNK,), jnp.float32),
            pltpu.VMEM((ECHUNK,), jnp.float32),
            pltpu.VMEM((ECHUNK, 16), jnp.float32),
            pltpu.VMEM((ECHUNK, 16), jnp.float32),
            pltpu.VMEM_SHARED((NPAD, 16), jnp.float32),
            pltpu.SemaphoreType.DMA,
            pltpu.SemaphoreType.DMA,
            pltpu.SemaphoreType.DMA,
            pltpu.SemaphoreType.DMA,
            pltpu.SemaphoreType.DMA,
            pltpu.SemaphoreType.DMA,
        ],
        compiler_params=pltpu.CompilerParams(use_tc_tiling_on_sc=False),
    )(qa, qb, rowp, colb, ewp)


_BNP = 2000


def _prep_body(u_ref, m_ref, d0_ref, d1_ref, qa_ref, qb_ref, dinv_ref):
    deg = d0_ref[...] + d1_ref[...] + 1.0
    dinv = lax.rsqrt(deg)
    u = u_ref[...]
    m = m_ref[...]
    qa_ref[...] = jnp.concatenate([u, m[:, :13]], axis=1) * dinv
    qb_ref[...] = jnp.concatenate(
        [m[:, 13:], jnp.zeros((_BNP, 11), jnp.float32)], axis=1) * dinv
    dinv_ref[...] = dinv


def _prep_call(user_x, movie_x, d0, d1):
    grid = (N_NODES // _BNP,)
    row_spec = lambda w: pl.BlockSpec((_BNP, w), lambda i: (i, 0))
    return pl.pallas_call(
        _prep_body,
        grid=grid,
        in_specs=[row_spec(3), row_spec(18), row_spec(1), row_spec(1)],
        out_specs=[row_spec(16), row_spec(16), row_spec(1)],
        out_shape=[jax.ShapeDtypeStruct((N_NODES, 16), jnp.float32),
                   jax.ShapeDtypeStruct((N_NODES, 16), jnp.float32),
                   jax.ShapeDtypeStruct((N_NODES, 1), jnp.float32)],
    )(user_x, movie_x, d0, d1)


_BNF = 6272


def _final_body(aa_ref, ab_ref, u_ref, m_ref, dinv_ref,
                wz_ref, bz_ref, w2_ref, b2_ref, o_ref):
    dinv = dinv_ref[...]
    agg = jnp.concatenate([aa_ref[...], ab_ref[...]], axis=1)
    x32 = jnp.concatenate(
        [u_ref[...], m_ref[...], jnp.zeros((_BNF, 11), jnp.float32)], axis=1)
    z = dinv * agg + (dinv * dinv) * x32
    hh = jnp.maximum(
        jnp.dot(z, wz_ref[...], preferred_element_type=jnp.float32)
        + bz_ref[...], 0.0)
    o_ref[...] = (jnp.dot(hh, w2_ref[...], preferred_element_type=jnp.float32)
                  + b2_ref[...])


def _final_call(aa, ab, user_x, movie_x, dinv, wz, bz, w2, b2):
    grid = (NPAD // _BNF,)
    row_spec = lambda w: pl.BlockSpec((_BNF, w), lambda i: (i, 0))
    full = lambda a, b: pl.BlockSpec((a, b), lambda i: (0, 0))
    return pl.pallas_call(
        _final_body,
        grid=grid,
        in_specs=[row_spec(16), row_spec(16), row_spec(3), row_spec(18),
                  row_spec(1), full(32, 128), full(1, 128), full(128, 1),
                  full(1, 1)],
        out_specs=row_spec(1),
        out_shape=jax.ShapeDtypeStruct((N_NODES, 1), jnp.float32),
    )(aa, ab, user_x, movie_x, dinv, wz, bz, w2, b2)


def kernel(user_x, movie_x, edge_index, edge_attr,
           W_u, b_u, W_m, b_m, W1, b1, W2, b2):
    E = edge_attr.shape[0]
    epad = _pad_up(E, 32 * DCHUNK)
    pad = epad - E

    row = edge_index[0].astype(jnp.int32)
    col = edge_index[1].astype(jnp.int32)
    zi = jnp.zeros((pad,), jnp.int32)
    rowp = jnp.concatenate([row, zi])
    colp = jnp.concatenate([col, zi])
    colb = colp.reshape(epad // KB, KB)
    ewp = jnp.concatenate([edge_attr, jnp.zeros((pad,), jnp.float32)])

    Wcat = jnp.zeros((32, 2 * HIDDEN), jnp.float32)
    Wcat = Wcat.at[0:3, 0:HIDDEN].set(W_u)
    Wcat = Wcat.at[3:21, HIDDEN:2 * HIDDEN].set(W_m)
    bcat = jnp.concatenate([b_u, b_m])
    Wz = Wcat @ W1
    bz = (bcat @ W1 + b1).reshape(1, -1)

    degp = _deg_call(colb, ewp)
    d0 = degp[0, :N_NODES].reshape(N_NODES, 1)
    d1 = degp[1, :N_NODES].reshape(N_NODES, 1)

    qa, qb, dinv = _prep_call(user_x, movie_x, d0, d1)
    acca, accb = _agg_call(qa, qb, rowp, colb, ewp)

    return _final_call(acca, accb, user_x, movie_x,
                       dinv, Wz, bz, W2, b2.reshape(1, 1))
